# baseline (speedup 1.0000x reference)
import functools

import jax
import jax.numpy as jnp
from jax import lax
from jax.experimental import pallas as pl
from jax.experimental.pallas import tpu as pltpu
from jax.experimental.pallas import tpu_sc as plsc

EPS = 1e-15

NC = 2
NS = 16
NW = NC * NS

WALK = 10
HID = 128
CHUNK = 128


def _make_gather(n_rows, chunks_per_worker):
    rows_per_worker = chunks_per_worker * CHUNK
    mesh = plsc.VectorSubcoreMesh(
        core_axis_name="c", subcore_axis_name="s",
        num_cores=NC, num_subcores=NS)

    nbuf = min(7, chunks_per_worker)
    glead = 3

    @functools.partial(
        pl.kernel,
        mesh=mesh,
        out_type=jax.ShapeDtypeStruct((n_rows, HID), jnp.float32),
        scratch_types=[
            pltpu.VMEM((chunks_per_worker, CHUNK), jnp.int32),
            [pltpu.VMEM((CHUNK, HID), jnp.float32) for _ in range(nbuf)],
            [pltpu.SemaphoreType.DMA for _ in range(nbuf)],
            [pltpu.SemaphoreType.DMA for _ in range(nbuf)],
        ],
    )
    def gather_k(table_hbm, idx_hbm, out_hbm, idx_v, bufs, gsems, wsems):
        wid = lax.axis_index("s") * NC + lax.axis_index("c")
        base = wid * rows_per_worker
        pltpu.sync_copy(idx_hbm.at[wid], idx_v)
        gath = [None] * chunks_per_worker
        wb = [None] * chunks_per_worker

        def writeback(j):
            gath[j].wait()
            wb[j] = pltpu.async_copy(
                bufs[j % nbuf], out_hbm.at[pl.ds(base + j * CHUNK, CHUNK)],
                wsems[j % nbuf])

        for j in range(chunks_per_worker):
            if j >= nbuf:
                wb[j - nbuf].wait()
            gath[j] = pltpu.async_copy(
                table_hbm.at[idx_v.at[j]], bufs[j % nbuf], gsems[j % nbuf])
            if j >= glead:
                writeback(j - glead)
        for j in range(max(0, chunks_per_worker - glead), chunks_per_worker):
            writeback(j)
        for j in range(max(0, chunks_per_worker - nbuf), chunks_per_worker):
            wb[j].wait()

    return gather_k


def _loss_body(nblk, pos_blocks, g_ref, wt_ref, b_ref, out_ref, acc_ref):
    i = pl.program_id(0)
    sign = jnp.where(i < pos_blocks, 1.0, -1.0)
    wt = wt_ref[...]
    bias = b_ref[...]
    t0 = jnp.tanh(g_ref[0].T)
    h0 = jnp.dot(wt, t0, preferred_element_type=jnp.float32) + bias
    rows = []
    for p in range(1, WALK):
        tp = jnp.tanh(g_ref[p].T)
        hp = jnp.dot(wt, tp, preferred_element_type=jnp.float32) + bias
        rows.append(jnp.sum(h0 * hp, axis=0, keepdims=True))
    s = jnp.concatenate(rows, axis=0)
    sig = 1.0 / (1.0 + jnp.exp(-sign * s))
    total = jnp.sum(-jnp.log(sig + EPS))

    @pl.when(i == 0)
    def _():
        acc_ref[0, 0] = 0.0

    acc_ref[0, 0] += total

    @pl.when(i == nblk - 1)
    def _():
        out_ref[0, 0] = acc_ref[0, 0]


def _loss_call(g3, Wt, bcol, nblk, pos_blocks, wb):
    return pl.pallas_call(
        functools.partial(_loss_body, nblk, pos_blocks),
        grid=(nblk,),
        in_specs=[
            pl.BlockSpec((WALK, wb, HID), lambda i: (0, i, 0)),
            pl.BlockSpec((HID, HID), lambda i: (0, 0)),
            pl.BlockSpec((HID, 1), lambda i: (0, 0)),
        ],
        out_specs=pl.BlockSpec(memory_space=pltpu.SMEM),
        out_shape=jax.ShapeDtypeStruct((1, 1), jnp.float32),
        scratch_shapes=[pltpu.SMEM((1, 1), jnp.float32)],
    )(g3, Wt, bcol)


def kernel(pos_rw, neg_rw, features, W, b):
    batch = pos_rw.shape[0]
    w_t = W.T
    bcol = b.reshape(HID, 1)
    wb = 2048

    cut = batch // 2
    slices = (
        (jnp.concatenate([pos_rw, neg_rw[:cut]], axis=0), batch // wb),
        (neg_rw[cut:], 0),
    )
    partials = []
    for rw, pos_blocks in slices:
        n = rw.shape[0]
        n_rows = n * WALK
        chunks_per_worker = n_rows // (NW * CHUNK)
        idx_flat = rw.T.reshape(-1).astype(jnp.int32)
        idx3 = idx_flat.reshape(NW, chunks_per_worker, CHUNK)
        gathered = _make_gather(n_rows, chunks_per_worker)(features, idx3)
        g3 = gathered.reshape(WALK, n, HID)
        partials.append(_loss_call(g3, w_t, bcol, n // wb, pos_blocks, wb))

    n_pairs = batch * (WALK - 1)
    total = partials[0][0, 0] + partials[1][0, 0]
    return total / jnp.float32(n_pairs)


# speedup vs baseline: 1.0112x; 1.0112x over previous
import functools

import jax
import jax.numpy as jnp
from jax import lax
from jax.experimental import pallas as pl
from jax.experimental.pallas import tpu as pltpu
from jax.experimental.pallas import tpu_sc as plsc

EPS = 1e-15

NC = 2
NS = 16
NW = NC * NS

WALK = 10
HID = 128
CHUNK = 128


def _make_gather(n_rows, chunks_per_worker):
    rows_per_worker = chunks_per_worker * CHUNK
    mesh = plsc.VectorSubcoreMesh(
        core_axis_name="c", subcore_axis_name="s",
        num_cores=NC, num_subcores=NS)

    assert chunks_per_worker % 2 == 0
    pairs = chunks_per_worker // 2
    nbuf = min(3, pairs)
    plead = 1

    @functools.partial(
        pl.kernel,
        mesh=mesh,
        out_type=jax.ShapeDtypeStruct((n_rows, HID), jnp.float32),
        scratch_types=[
            pltpu.VMEM((chunks_per_worker, CHUNK), jnp.int32),
            [pltpu.VMEM((2 * CHUNK, HID), jnp.float32) for _ in range(nbuf)],
            [pltpu.SemaphoreType.DMA for _ in range(nbuf)],
            [pltpu.SemaphoreType.DMA for _ in range(nbuf)],
            [pltpu.SemaphoreType.DMA for _ in range(nbuf)],
        ],
    )
    def gather_k(table_hbm, idx_hbm, out_hbm, idx_v, bufs, g0sems, g1sems,
                 wsems):
        wid = lax.axis_index("s") * NC + lax.axis_index("c")
        base = wid * rows_per_worker
        pltpu.sync_copy(idx_hbm.at[wid], idx_v)
        gath = [None] * chunks_per_worker
        wb = [None] * pairs

        def writeback(p):
            gath[2 * p].wait()
            gath[2 * p + 1].wait()
            wb[p] = pltpu.async_copy(
                bufs[p % nbuf],
                out_hbm.at[pl.ds(base + 2 * p * CHUNK, 2 * CHUNK)],
                wsems[p % nbuf])

        for p in range(pairs):
            if p >= nbuf:
                wb[p - nbuf].wait()
            buf = bufs[p % nbuf]
            gath[2 * p] = pltpu.async_copy(
                table_hbm.at[idx_v.at[2 * p]],
                buf.at[pl.ds(0, CHUNK)], g0sems[p % nbuf])
            gath[2 * p + 1] = pltpu.async_copy(
                table_hbm.at[idx_v.at[2 * p + 1]],
                buf.at[pl.ds(CHUNK, CHUNK)], g1sems[p % nbuf])
            if p >= plead:
                writeback(p - plead)
        for p in range(max(0, pairs - plead), pairs):
            writeback(p)
        for p in range(max(0, pairs - nbuf), pairs):
            wb[p].wait()

    return gather_k


def _loss_body(nblk, sign, g_ref, wt_ref, b_ref, out_ref, acc_ref):
    i = pl.program_id(0)
    wt = wt_ref[...]
    bias = b_ref[...]
    t0 = jnp.tanh(g_ref[0].T)
    h0 = jnp.dot(wt, t0, preferred_element_type=jnp.float32) + bias
    rows = []
    for p in range(1, WALK):
        tp = jnp.tanh(g_ref[p].T)
        hp = jnp.dot(wt, tp, preferred_element_type=jnp.float32) + bias
        rows.append(jnp.sum(h0 * hp, axis=0, keepdims=True))
    s = jnp.concatenate(rows, axis=0)
    sig = 1.0 / (1.0 + jnp.exp(-sign * s))
    total = jnp.sum(-jnp.log(sig + EPS))

    @pl.when(i == 0)
    def _():
        acc_ref[0, 0] = 0.0

    acc_ref[0, 0] += total

    @pl.when(i == nblk - 1)
    def _():
        out_ref[0, 0] = acc_ref[0, 0]


def _loss_call(g3, Wt, bcol, nblk, sign, wb):
    return pl.pallas_call(
        functools.partial(_loss_body, nblk, sign),
        grid=(nblk,),
        in_specs=[
            pl.BlockSpec((WALK, wb, HID), lambda i: (0, i, 0)),
            pl.BlockSpec((HID, HID), lambda i: (0, 0)),
            pl.BlockSpec((HID, 1), lambda i: (0, 0)),
        ],
        out_specs=pl.BlockSpec(memory_space=pltpu.SMEM),
        out_shape=jax.ShapeDtypeStruct((1, 1), jnp.float32),
        scratch_shapes=[pltpu.SMEM((1, 1), jnp.float32)],
    )(g3, Wt, bcol)


def kernel(pos_rw, neg_rw, features, W, b):
    batch = pos_rw.shape[0]
    w_t = W.T
    bcol = b.reshape(HID, 1)
    wb = 2048

    slices = (
        (pos_rw, 1.0),
        (neg_rw, -1.0),
    )
    partials = []
    for rw, sign in slices:
        n = rw.shape[0]
        n_rows = n * WALK
        chunks_per_worker = n_rows // (NW * CHUNK)
        idx_flat = rw.T.reshape(-1).astype(jnp.int32)
        idx3 = idx_flat.reshape(NW, chunks_per_worker, CHUNK)
        gathered = _make_gather(n_rows, chunks_per_worker)(features, idx3)
        g3 = gathered.reshape(WALK, n, HID)
        partials.append(_loss_call(g3, w_t, bcol, n // wb, sign, wb))

    n_pairs = batch * (WALK - 1)
    total = partials[0][0, 0] + partials[1][0, 0]
    return total / jnp.float32(n_pairs)


# speedup vs baseline: 1.0647x; 1.0530x over previous
import functools

import jax
import jax.numpy as jnp
from jax import lax
from jax.experimental import pallas as pl
from jax.experimental.pallas import tpu as pltpu
from jax.experimental.pallas import tpu_sc as plsc

EPS = 1e-15

NC = 2
NS = 16
NW = NC * NS

WALK = 10
HID = 128
CHUNK = 128


def _make_gather(n_rows, chunks_per_worker):
    rows_per_worker = chunks_per_worker * CHUNK
    mesh = plsc.VectorSubcoreMesh(
        core_axis_name="c", subcore_axis_name="s",
        num_cores=NC, num_subcores=NS)

    nbuf = min(7, chunks_per_worker)
    glead = 5

    @functools.partial(
        pl.kernel,
        mesh=mesh,
        out_type=jax.ShapeDtypeStruct((n_rows, HID), jnp.float32),
        scratch_types=[
            pltpu.VMEM((chunks_per_worker, CHUNK), jnp.int32),
            [pltpu.VMEM((CHUNK, HID), jnp.float32) for _ in range(nbuf)],
            [pltpu.SemaphoreType.DMA for _ in range(nbuf)],
            [pltpu.SemaphoreType.DMA for _ in range(nbuf)],
        ],
    )
    def gather_k(table_hbm, idx_hbm, out_hbm, idx_v, bufs, gsems, wsems):
        wid = lax.axis_index("s") * NC + lax.axis_index("c")
        base = wid * rows_per_worker
        pltpu.sync_copy(idx_hbm.at[wid], idx_v)
        gath = [None] * chunks_per_worker
        wb = [None] * chunks_per_worker

        def writeback(j):
            gath[j].wait()
            wb[j] = pltpu.async_copy(
                bufs[j % nbuf], out_hbm.at[pl.ds(base + j * CHUNK, CHUNK)],
                wsems[j % nbuf])

        for j in range(chunks_per_worker):
            if j >= nbuf:
                wb[j - nbuf].wait()
            gath[j] = pltpu.async_copy(
                table_hbm.at[idx_v.at[j]], bufs[j % nbuf], gsems[j % nbuf])
            if j >= glead:
                writeback(j - glead)
        for j in range(max(0, chunks_per_worker - glead), chunks_per_worker):
            writeback(j)
        for j in range(max(0, chunks_per_worker - nbuf), chunks_per_worker):
            wb[j].wait()

    return gather_k


def _loss_body(nblk, sign, g_ref, wt_ref, b_ref, out_ref, acc_ref):
    i = pl.program_id(0)
    wt = wt_ref[...]
    bias = b_ref[...]
    t0 = jnp.tanh(g_ref[0].T)
    h0 = jnp.dot(wt, t0, preferred_element_type=jnp.float32) + bias
    rows = []
    for p in range(1, WALK):
        tp = jnp.tanh(g_ref[p].T)
        hp = jnp.dot(wt, tp, preferred_element_type=jnp.float32) + bias
        rows.append(jnp.sum(h0 * hp, axis=0, keepdims=True))
    s = jnp.concatenate(rows, axis=0)
    sig = 1.0 / (1.0 + jnp.exp(-sign * s))
    total = jnp.sum(-jnp.log(sig + EPS))

    @pl.when(i == 0)
    def _():
        acc_ref[0, 0] = 0.0

    acc_ref[0, 0] += total

    @pl.when(i == nblk - 1)
    def _():
        out_ref[0, 0] = acc_ref[0, 0]


def _loss_call(g3, Wt, bcol, nblk, sign, wb):
    return pl.pallas_call(
        functools.partial(_loss_body, nblk, sign),
        grid=(nblk,),
        in_specs=[
            pl.BlockSpec((WALK, wb, HID), lambda i: (0, i, 0)),
            pl.BlockSpec((HID, HID), lambda i: (0, 0)),
            pl.BlockSpec((HID, 1), lambda i: (0, 0)),
        ],
        out_specs=pl.BlockSpec(memory_space=pltpu.SMEM),
        out_shape=jax.ShapeDtypeStruct((1, 1), jnp.float32),
        scratch_shapes=[pltpu.SMEM((1, 1), jnp.float32)],
    )(g3, Wt, bcol)


def kernel(pos_rw, neg_rw, features, W, b):
    batch = pos_rw.shape[0]
    w_t = W.T
    bcol = b.reshape(HID, 1)
    wb = 2048

    slices = (
        (pos_rw, 1.0),
        (neg_rw, -1.0),
    )
    partials = []
    for rw, sign in slices:
        n = rw.shape[0]
        n_rows = n * WALK
        chunks_per_worker = n_rows // (NW * CHUNK)
        idx_flat = rw.T.reshape(-1).astype(jnp.int32)
        idx3 = idx_flat.reshape(NW, chunks_per_worker, CHUNK)
        gathered = _make_gather(n_rows, chunks_per_worker)(features, idx3)
        g3 = gathered.reshape(WALK, n, HID)
        partials.append(_loss_call(g3, w_t, bcol, n // wb, sign, wb))

    n_pairs = batch * (WALK - 1)
    total = partials[0][0, 0] + partials[1][0, 0]
    return total / jnp.float32(n_pairs)
